# baseline (speedup 1.0000x reference)
import jax
import jax.numpy as jnp
from jax.experimental import pallas as pl
from jax.experimental.pallas import tpu as pltpu

B = 16384
CLASS_NUM = 2
FEATURE_DIM = 1024
BLK = 4096
NBLK = B // BLK


def _body(tag_ref, feat_ref, center_ref, out_ref, acc_ref):
    i = pl.program_id(0)
    t = tag_ref[0, 0, :]
    f = feat_ref[...]
    cen = center_ref[...]
    q = jnp.sum(f * f, axis=1)
    P = jax.lax.dot_general(f, cen, (((1,), (1,)), ((), ())),
                            preferred_element_type=jnp.float32)
    cs = jnp.sum(cen * cen, axis=1)
    tf = t.astype(jnp.float32)
    p0 = P[:, 0]
    p1 = P[:, 1]
    p = p0 + tf * (p1 - p0)
    csel = cs[0] + tf * (cs[1] - cs[0])
    d2 = q - 2.0 * p + csel
    d = jnp.sqrt(jnp.maximum(d2, 0.0))
    s1 = jnp.sum(d * tf)
    s_all = jnp.sum(d)
    n1 = jnp.sum(tf)

    @pl.when(i == 0)
    def _():
        acc_ref[0] = 0.0
        acc_ref[1] = 0.0
        acc_ref[2] = 0.0

    acc_ref[0] += s_all - s1
    acc_ref[1] += s1
    acc_ref[2] += n1

    @pl.when(i == NBLK - 1)
    def _():
        s0_t = acc_ref[0]
        s1_t = acc_ref[1]
        n1_t = acc_ref[2]
        n0_t = jnp.float32(B) - n1_t
        l0 = jnp.where(n0_t > 0, s0_t / jnp.maximum(n0_t, 1.0), 0.0)
        l1 = jnp.where(n1_t > 0, s1_t / jnp.maximum(n1_t, 1.0), 0.0)
        out_ref[0] = l0 + l1


def kernel(tag, feature, center):
    tag3 = tag.reshape(NBLK, 1, BLK)
    out = pl.pallas_call(
        _body,
        grid=(NBLK,),
        in_specs=[
            pl.BlockSpec((1, 1, BLK), lambda i: (i, 0, 0)),
            pl.BlockSpec((BLK, FEATURE_DIM), lambda i: (i, 0)),
            pl.BlockSpec((CLASS_NUM, FEATURE_DIM), lambda i: (0, 0)),
        ],
        out_specs=pl.BlockSpec(memory_space=pltpu.MemorySpace.SMEM),
        out_shape=jax.ShapeDtypeStruct((1,), jnp.float32),
        scratch_shapes=[pltpu.SMEM((3,), jnp.float32)],
    )(tag3, feature, center)
    return out[0]
